# baseline (speedup 1.0000x reference)
import jax
import jax.numpy as jnp
from jax.experimental import pallas as pl
from jax.experimental.pallas import tpu as pltpu

B = 16
C = 3
IMG = 384
P = 16
G = IMG // P
N = G * G
E = 4
TOPK = 2
EXPERT_DIM = 1024
HIDDEN = 1024
PATCH_DIM = C * P * P


def _fuse_kernel(w1_ref, w2_ref, b1_ref, b2_ref, wc_ref, bc_ref):
    w2 = w2_ref[0]
    wc = jnp.dot(w1_ref[0], w2, preferred_element_type=jnp.float32)
    wc_ref[0] = wc.astype(jnp.bfloat16)
    bc_ref[0] = jnp.dot(b1_ref[0], w2, preferred_element_type=jnp.float32) + b2_ref[0]


def _routed_matmul_kernel(sel_ref, rw_ref, x_ref, wc_ref, bc_ref, out_ref,
                          p_scr):
    i = pl.program_id(0)
    nb = pl.num_programs(0)

    xb = x_ref[0].astype(jnp.bfloat16)
    p_scr[i % 2] = xb.reshape(C, G, P, G, P).transpose(1, 3, 0, 2, 4).reshape(
        N, PATCH_DIM)

    j = jnp.maximum(i - 1, 0)
    s0 = sel_ref[j, 0]
    s1 = sel_ref[j, 1]
    w0 = rw_ref[j, 0]
    w1 = rw_ref[j, 1]

    def coef(e):
        c0 = jnp.where(s0 == e, w0, jnp.float32(0.0))
        c1 = jnp.where(s1 == e, w1, jnp.float32(0.0))
        return c0 + c1

    cs = [coef(e) for e in range(E)]
    w_eff = cs[0].astype(jnp.bfloat16) * wc_ref[0]
    for e in range(1, E):
        w_eff = w_eff + cs[e].astype(jnp.bfloat16) * wc_ref[e]
    bias = cs[0] * bc_ref[0]
    for e in range(1, E):
        bias = bias + cs[e] * bc_ref[e]

    out_ref[0] = jnp.dot(p_scr[j % 2], w_eff,
                         preferred_element_type=jnp.float32) + bias


def kernel(x, selected_experts, routing_weights, W_patch, b_patch, W_proj, b_proj):
    xb = x.shape[0]

    w_comb, b_comb = pl.pallas_call(
        _fuse_kernel,
        grid=(E,),
        in_specs=[
            pl.BlockSpec((1, PATCH_DIM, EXPERT_DIM), lambda e: (e, 0, 0)),
            pl.BlockSpec((1, EXPERT_DIM, HIDDEN), lambda e: (e, 0, 0)),
            pl.BlockSpec((1, 1, EXPERT_DIM), lambda e: (e, 0, 0)),
            pl.BlockSpec((1, 1, HIDDEN), lambda e: (e, 0, 0)),
        ],
        out_specs=[
            pl.BlockSpec((1, PATCH_DIM, HIDDEN), lambda e: (e, 0, 0)),
            pl.BlockSpec((1, 1, HIDDEN), lambda e: (e, 0, 0)),
        ],
        out_shape=[
            jax.ShapeDtypeStruct((E, PATCH_DIM, HIDDEN), jnp.bfloat16),
            jax.ShapeDtypeStruct((E, 1, HIDDEN), jnp.float32),
        ],
    )(W_patch, W_proj, b_patch.reshape(E, 1, EXPERT_DIM),
      b_proj.reshape(E, 1, HIDDEN))

    out = pl.pallas_call(
        _routed_matmul_kernel,
        grid_spec=pltpu.PrefetchScalarGridSpec(
            num_scalar_prefetch=2,
            grid=(xb + 1,),
            in_specs=[
                pl.BlockSpec((1, C, IMG, IMG),
                             lambda i, sel, rw: (jnp.minimum(i, B - 1), 0, 0, 0)),
                pl.BlockSpec((E, PATCH_DIM, HIDDEN), lambda i, sel, rw: (0, 0, 0)),
                pl.BlockSpec((E, 1, HIDDEN), lambda i, sel, rw: (0, 0, 0)),
            ],
            out_specs=pl.BlockSpec(
                (1, N, HIDDEN),
                lambda i, sel, rw: (jnp.maximum(i - 1, 0), 0, 0)),
            scratch_shapes=[
                pltpu.VMEM((2, N, PATCH_DIM), jnp.bfloat16),
            ],
        ),
        out_shape=jax.ShapeDtypeStruct((xb, N, HIDDEN), jnp.float32),
    )(selected_experts.astype(jnp.int32), routing_weights, x, w_comb,
      b_comb)
    return out
